# baseline (speedup 1.0000x reference)
import jax
import jax.numpy as jnp
from jax.experimental import pallas as pl
from jax.experimental.pallas import tpu as pltpu

IMG = 512
KK = 1000
CC = 0.001 / KK


def _body(steps_ref, x_ref, o_ref):
    i = pl.program_id(0)
    step = steps_ref[i]
    idx = jnp.where(step == 0, KK - 1, step - 1)
    val = CC * idx.astype(jnp.float32)
    rows = jax.lax.broadcasted_iota(jnp.int32, (1, IMG, IMG), 1)
    cols = jax.lax.broadcasted_iota(jnp.int32, (1, IMG, IMG), 2)
    diag = rows == cols
    o_ref[...] = x_ref[...] + jnp.where(diag, val, 0.0).astype(x_ref.dtype)


def kernel(x, fwd_steps):
    b = x.shape[0]
    grid_spec = pltpu.PrefetchScalarGridSpec(
        num_scalar_prefetch=1,
        grid=(b,),
        in_specs=[pl.BlockSpec((1, IMG, IMG), lambda i, steps: (i, 0, 0))],
        out_specs=pl.BlockSpec((1, IMG, IMG), lambda i, steps: (i, 0, 0)),
    )
    return pl.pallas_call(
        _body,
        grid_spec=grid_spec,
        out_shape=jax.ShapeDtypeStruct(x.shape, x.dtype),
    )(fwd_steps.astype(jnp.int32), x)


# speedup vs baseline: 1.5129x; 1.5129x over previous
import jax
import jax.numpy as jnp
from jax.experimental import pallas as pl
from jax.experimental.pallas import tpu as pltpu

IMG = 512
KK = 1000
CC = 0.001 / KK
NB = 8


def _body(steps_ref, x_ref, eye_ref, o_ref):
    i = pl.program_id(0)
    eye = eye_ref[...]
    for s in range(NB):
        step = steps_ref[i * NB + s]
        idx = jnp.where(step == 0, KK - 1, step - 1)
        val = CC * idx.astype(jnp.float32)
        o_ref[s] = x_ref[s] + val * eye


def kernel(x, fwd_steps):
    b = x.shape[0]
    eye = jnp.eye(IMG, dtype=x.dtype)
    grid_spec = pltpu.PrefetchScalarGridSpec(
        num_scalar_prefetch=1,
        grid=(b // NB,),
        in_specs=[
            pl.BlockSpec((NB, IMG, IMG), lambda i, steps: (i, 0, 0)),
            pl.BlockSpec((IMG, IMG), lambda i, steps: (0, 0)),
        ],
        out_specs=pl.BlockSpec((NB, IMG, IMG), lambda i, steps: (i, 0, 0)),
    )
    return pl.pallas_call(
        _body,
        grid_spec=grid_spec,
        out_shape=jax.ShapeDtypeStruct(x.shape, x.dtype),
    )(fwd_steps.astype(jnp.int32), x, eye)


# speedup vs baseline: 1.5423x; 1.0195x over previous
import jax
import jax.numpy as jnp
from jax.experimental import pallas as pl
from jax.experimental.pallas import tpu as pltpu

IMG = 512
KK = 1000
CC = 0.001 / KK
NB = 8


def _body(steps_ref, x_ref, o_ref, eye_ref):
    i = pl.program_id(0)

    @pl.when(i == 0)
    def _fill_eye():
        rows = jax.lax.broadcasted_iota(jnp.int32, (IMG, IMG), 0)
        cols = jax.lax.broadcasted_iota(jnp.int32, (IMG, IMG), 1)
        eye_ref[...] = jnp.where(rows == cols, 1.0, 0.0).astype(jnp.float32)

    eye = eye_ref[...]
    for s in range(NB):
        step = steps_ref[i * NB + s]
        idx = jnp.where(step == 0, KK - 1, step - 1)
        val = CC * idx.astype(jnp.float32)
        o_ref[s] = x_ref[s] + val * eye


def kernel(x, fwd_steps):
    b = x.shape[0]
    grid_spec = pltpu.PrefetchScalarGridSpec(
        num_scalar_prefetch=1,
        grid=(b // NB,),
        in_specs=[pl.BlockSpec((NB, IMG, IMG), lambda i, steps: (i, 0, 0))],
        out_specs=pl.BlockSpec((NB, IMG, IMG), lambda i, steps: (i, 0, 0)),
        scratch_shapes=[pltpu.VMEM((IMG, IMG), jnp.float32)],
    )
    return pl.pallas_call(
        _body,
        grid_spec=grid_spec,
        out_shape=jax.ShapeDtypeStruct(x.shape, x.dtype),
    )(fwd_steps.astype(jnp.int32), x)
